# baseline (speedup 1.0000x reference)
import functools

import jax
import jax.numpy as jnp
from jax import lax
from jax.experimental import pallas as pl
from jax.experimental.pallas import tpu as pltpu
from jax.experimental.pallas import tpu_sc as plsc

B = 16384
NF = 64
H1 = 256
NROWS = 1000000

NC = 2
NS = 16
NW = NC * NS
BPW = B // NW

TBLK = 40960
HB = TBLK // 2
NGRID = (NROWS + TBLK - 1) // TBLK
NFUSED = NGRID * HB


CXLU = 13696


def _transpose_body(t_ref, eye_ref, o_ref):
    for half, lo in ((0, 0), (1, HB)):
        x = t_ref[:, pl.ds(lo, HB)]
        o_ref[:CXLU, pl.ds(half * NF, NF)] = x[:, :CXLU].T
        o_ref[CXLU:, pl.ds(half * NF, NF)] = lax.dot_general(
            x[:, CXLU:], eye_ref[...], (((0,), (0,)), ((), ())),
            preferred_element_type=jnp.float32)


def _transpose(tT, eye):
    return pl.pallas_call(
        _transpose_body,
        grid=(NGRID,),
        in_specs=[
            pl.BlockSpec((NF, TBLK), lambda i: (0, i)),
            pl.BlockSpec((NF, NF), lambda i: (0, 0)),
        ],
        out_specs=pl.BlockSpec((HB, 2 * NF), lambda i: (i, 0)),
        out_shape=jax.ShapeDtypeStruct((NFUSED, 2 * NF), jnp.float32),
    )(tT, eye)


def _sc_gather(idx2, table2):
    mesh = plsc.VectorSubcoreMesh(core_axis_name="c", subcore_axis_name="s")

    @functools.partial(
        pl.kernel,
        out_type=jax.ShapeDtypeStruct((B, 2 * NF), jnp.float32),
        mesh=mesh,
        scratch_types=[
            pltpu.VMEM((BPW,), jnp.int32),
            pltpu.VMEM((BPW, 2 * NF), jnp.float32),
            pltpu.SemaphoreType.DMA,
        ],
    )
    def k(idx_hbm, table_hbm, out_hbm, idx_ref, rows, sem):
        wid = lax.axis_index("s") * NC + lax.axis_index("c")
        base = wid * BPW
        pltpu.sync_copy(idx_hbm.at[wid], idx_ref)

        def group(t, _):
            v16 = idx_ref[pl.ds(t * 16, 16)]
            for l in range(16):
                s = v16[l]
                pltpu.async_copy(
                    table_hbm.at[s], rows.at[t * 16 + l], sem)
            return 0

        lax.fori_loop(0, BPW // 16, group, 0)

        def drain(j, _):
            pltpu.make_async_copy(table_hbm.at[0], rows.at[0], sem).wait()
            return 0

        lax.fori_loop(0, BPW, drain, 0)
        pltpu.sync_copy(rows, out_hbm.at[pl.ds(base, BPW)])

    return k(idx2, table2)


BS = 2048


def _mlp_body(xu_ref, xi_ref, uh_ref, ih_ref, w1u_ref, w1i_ref,
              b1_ref, w2t_ref, b2_ref, o_ref):
    xu = xu_ref[...]
    xi = xi_ref[...]
    u_emb = jnp.where(uh_ref[...] != 0, xu[:, NF:], xu[:, :NF])
    i_emb = jnp.where(ih_ref[...] != 0, xi[:, NF:], xi[:, :NF])
    h = (
        jnp.dot(u_emb, w1u_ref[...], preferred_element_type=jnp.float32)
        + jnp.dot(i_emb, w1i_ref[...], preferred_element_type=jnp.float32)
        + b1_ref[...]
    )
    h = jnp.maximum(h, 0.0)
    o_ref[...] = jnp.sum(h * w2t_ref[...], axis=1, keepdims=True) + b2_ref[...]


def _mlp(xu, xi, uh, ih, W1u, W1i, b1, W2t, b2):
    return pl.pallas_call(
        _mlp_body,
        grid=(B // BS,),
        in_specs=[
            pl.BlockSpec((BS, 2 * NF), lambda i: (i, 0)),
            pl.BlockSpec((BS, 2 * NF), lambda i: (i, 0)),
            pl.BlockSpec((BS, 1), lambda i: (i, 0)),
            pl.BlockSpec((BS, 1), lambda i: (i, 0)),
            pl.BlockSpec((NF, H1), lambda i: (0, 0)),
            pl.BlockSpec((NF, H1), lambda i: (0, 0)),
            pl.BlockSpec((1, H1), lambda i: (0, 0)),
            pl.BlockSpec((1, H1), lambda i: (0, 0)),
            pl.BlockSpec((1, 1), lambda i: (0, 0)),
        ],
        out_specs=pl.BlockSpec((BS, 1), lambda i: (i, 0)),
        out_shape=jax.ShapeDtypeStruct((B, 1), jnp.float32),
    )(xu, xi, uh, ih, W1u, W1i, b1, W2t, b2)


@jax.jit
def kernel(users, items, user_table, item_table, W1, b1, W2, b2):
    eye = jnp.eye(NF, dtype=jnp.float32)
    ut2 = _transpose(user_table.T, eye)
    it2 = _transpose(item_table.T, eye)

    def fuse_idx(r):
        g = r // TBLK
        w = r % TBLK
        return g * HB + w % HB, w // HB

    uF, uh = fuse_idx(users)
    iF, ih = fuse_idx(items)
    xu = _sc_gather(uF.reshape(NW, BPW), ut2)
    xi = _sc_gather(iF.reshape(NW, BPW), it2)
    W1u = W1[:NF]
    W1i = W1[NF:]
    return _mlp(xu, xi, uh.reshape(B, 1), ih.reshape(B, 1), W1u, W1i,
                b1.reshape(1, H1), W2.reshape(1, H1), b2.reshape(1, 1))


# speedup vs baseline: 1.0078x; 1.0078x over previous
import functools

import jax
import jax.numpy as jnp
from jax import lax
from jax.experimental import pallas as pl
from jax.experimental.pallas import tpu as pltpu
from jax.experimental.pallas import tpu_sc as plsc

B = 16384
NF = 64
H1 = 256
NROWS = 1000000

NC = 2
NS = 16
NW = NC * NS
BPW = B // NW

TBLK = 32768
HB = TBLK // 2
NGRID = (NROWS + TBLK - 1) // TBLK
NFUSED = NGRID * HB


CXLU = 10752


def _transpose_body(t_ref, eye_ref, o_ref):
    for half, lo in ((0, 0), (1, HB)):
        x = t_ref[:, pl.ds(lo, HB)]
        o_ref[:CXLU, pl.ds(half * NF, NF)] = x[:, :CXLU].T
        o_ref[CXLU:, pl.ds(half * NF, NF)] = lax.dot_general(
            x[:, CXLU:], eye_ref[...], (((0,), (0,)), ((), ())),
            preferred_element_type=jnp.float32)


def _transpose(tT, eye):
    return pl.pallas_call(
        _transpose_body,
        grid=(NGRID,),
        in_specs=[
            pl.BlockSpec((NF, TBLK), lambda i: (0, i)),
            pl.BlockSpec((NF, NF), lambda i: (0, 0)),
        ],
        out_specs=pl.BlockSpec((HB, 2 * NF), lambda i: (i, 0)),
        out_shape=jax.ShapeDtypeStruct((NFUSED, 2 * NF), jnp.float32),
    )(tT, eye)


def _sc_gather(idx2, table2):
    mesh = plsc.VectorSubcoreMesh(core_axis_name="c", subcore_axis_name="s")

    @functools.partial(
        pl.kernel,
        out_type=jax.ShapeDtypeStruct((B, 2 * NF), jnp.float32),
        mesh=mesh,
        scratch_types=[
            pltpu.VMEM((BPW,), jnp.int32),
            pltpu.VMEM((BPW, 2 * NF), jnp.float32),
            pltpu.SemaphoreType.DMA,
        ],
    )
    def k(idx_hbm, table_hbm, out_hbm, idx_ref, rows, sem):
        wid = lax.axis_index("s") * NC + lax.axis_index("c")
        base = wid * BPW
        pltpu.sync_copy(idx_hbm.at[wid], idx_ref)

        def group(t, _):
            v16 = idx_ref[pl.ds(t * 16, 16)]
            for l in range(16):
                s = v16[l]
                pltpu.async_copy(
                    table_hbm.at[s], rows.at[t * 16 + l], sem)
            return 0

        lax.fori_loop(0, BPW // 16, group, 0)

        def drain(j, _):
            pltpu.make_async_copy(table_hbm.at[0], rows.at[0], sem).wait()
            return 0

        lax.fori_loop(0, BPW, drain, 0)
        pltpu.sync_copy(rows, out_hbm.at[pl.ds(base, BPW)])

    return k(idx2, table2)


BS = 2048


def _mlp_body(xu_ref, xi_ref, uh_ref, ih_ref, w1u_ref, w1i_ref,
              b1_ref, w2t_ref, b2_ref, o_ref):
    xu = xu_ref[...]
    xi = xi_ref[...]
    u_emb = jnp.where(uh_ref[...] != 0, xu[:, NF:], xu[:, :NF])
    i_emb = jnp.where(ih_ref[...] != 0, xi[:, NF:], xi[:, :NF])
    h = (
        jnp.dot(u_emb, w1u_ref[...], preferred_element_type=jnp.float32)
        + jnp.dot(i_emb, w1i_ref[...], preferred_element_type=jnp.float32)
        + b1_ref[...]
    )
    h = jnp.maximum(h, 0.0)
    o_ref[...] = jnp.sum(h * w2t_ref[...], axis=1, keepdims=True) + b2_ref[...]


def _mlp(xu, xi, uh, ih, W1u, W1i, b1, W2t, b2):
    return pl.pallas_call(
        _mlp_body,
        grid=(B // BS,),
        in_specs=[
            pl.BlockSpec((BS, 2 * NF), lambda i: (i, 0)),
            pl.BlockSpec((BS, 2 * NF), lambda i: (i, 0)),
            pl.BlockSpec((BS, 1), lambda i: (i, 0)),
            pl.BlockSpec((BS, 1), lambda i: (i, 0)),
            pl.BlockSpec((NF, H1), lambda i: (0, 0)),
            pl.BlockSpec((NF, H1), lambda i: (0, 0)),
            pl.BlockSpec((1, H1), lambda i: (0, 0)),
            pl.BlockSpec((1, H1), lambda i: (0, 0)),
            pl.BlockSpec((1, 1), lambda i: (0, 0)),
        ],
        out_specs=pl.BlockSpec((BS, 1), lambda i: (i, 0)),
        out_shape=jax.ShapeDtypeStruct((B, 1), jnp.float32),
    )(xu, xi, uh, ih, W1u, W1i, b1, W2t, b2)


@jax.jit
def kernel(users, items, user_table, item_table, W1, b1, W2, b2):
    eye = jnp.eye(NF, dtype=jnp.float32)
    ut2 = _transpose(user_table.T, eye)
    it2 = _transpose(item_table.T, eye)

    def fuse_idx(r):
        g = r // TBLK
        w = r % TBLK
        return g * HB + w % HB, w // HB

    uF, uh = fuse_idx(users)
    iF, ih = fuse_idx(items)
    xu = _sc_gather(uF.reshape(NW, BPW), ut2)
    xi = _sc_gather(iF.reshape(NW, BPW), it2)
    W1u = W1[:NF]
    W1i = W1[NF:]
    return _mlp(xu, xi, uh.reshape(B, 1), ih.reshape(B, 1), W1u, W1i,
                b1.reshape(1, H1), W2.reshape(1, H1), b2.reshape(1, 1))
